# baseline (speedup 1.0000x reference)
import functools

import jax
import jax.numpy as jnp
from jax import lax
from jax.experimental import pallas as pl
from jax.experimental.pallas import tpu as pltpu
from jax.experimental.pallas import tpu_sc as plsc

CAP = 100000
PAD = 102400
D = 64
K = 32
B = 1024
EPS = 1e-08

QT = 128
KB = 4096
CHUNK = 128
NCHUNK = PAD // CHUNK
CPB = KB // CHUNK
MID = 1024
NMID = PAD // MID
MPB = KB // MID

NW = 32
RPW = B // NW
HOTCAP = 640
CCAP = 512
CBUF = CCAP + 32
INF = float("inf")
BIGF = 1e30


def _dist_kernel(q_ref, k_ref, qn_ref, kn_ref, d_ref, m_ref):
    q = q_ref[...]
    k = k_ref[...]
    sq = qn_ref[...] + kn_ref[...] - 2.0 * lax.dot_general(
        q, k, (((1,), (1,)), ((), ())),
        preferred_element_type=jnp.float32,
        precision=lax.Precision.DEFAULT)
    s = jnp.maximum(sq, 1e-12)
    d_ref[...] = s
    m_ref[0, ...] = jnp.min(s.reshape(QT, CPB, CHUNK), axis=2)


def _tc_stage(queries, keys_p):
    qn = jnp.sum(queries * queries, axis=1, keepdims=True)
    kn = jnp.sum(keys_p * keys_p, axis=1, keepdims=True).T
    grid = (B // QT, PAD // KB)
    return pl.pallas_call(
        _dist_kernel,
        grid=grid,
        in_specs=[
            pl.BlockSpec((QT, D), lambda i, j: (i, 0)),
            pl.BlockSpec((KB, D), lambda i, j: (j, 0)),
            pl.BlockSpec((QT, 1), lambda i, j: (i, 0)),
            pl.BlockSpec((1, KB), lambda i, j: (0, j)),
        ],
        out_specs=[
            pl.BlockSpec((QT, KB), lambda i, j: (i, j)),
            pl.BlockSpec((1, QT, CPB), lambda i, j: (j, i, 0)),
        ],
        out_shape=[
            jax.ShapeDtypeStruct((B, PAD), jnp.float32),
            jax.ShapeDtypeStruct((PAD // KB, B, CPB), jnp.float32),
        ],
    )(queries, keys_p, qn, kn)


def _t2_kernel(m_ref, t2_ref):
    x = m_ref[...]
    m = jnp.min(x, axis=(0, 2), keepdims=True)
    for _ in range(K - 1):
        x = jnp.where(x == m, INF, x)
        m = jnp.min(x, axis=(0, 2), keepdims=True)
    t2_ref[...] = m[0]


def _t2_stage(m3):
    return pl.pallas_call(
        _t2_kernel,
        grid=(B // QT,),
        in_specs=[pl.BlockSpec((PAD // KB, QT, CPB), lambda i: (0, i, 0))],
        out_specs=pl.BlockSpec((QT, 1), lambda i: (i, 0)),
        out_shape=jax.ShapeDtypeStruct((B, 1), jnp.float32),
    )(m3)


def _dyn_gather(x, idx16):
    return lax.gather(
        x, idx16[:, None],
        lax.GatherDimensionNumbers(
            offset_dims=(), collapsed_slice_dims=(0,), start_index_map=(0,)),
        (1,), mode=lax.GatherScatterMode.PROMISE_IN_BOUNDS)


_MESH = plsc.VectorSubcoreMesh(core_axis_name="c", subcore_axis_name="s")


@functools.partial(
    pl.kernel,
    mesh=_MESH,
    compiler_params=pltpu.CompilerParams(needs_layout_passes=False),
    out_type=jax.ShapeDtypeStruct((B,), jnp.float32),
    scratch_types=[
        pltpu.VMEM((2, 32, 128), jnp.float32),
        pltpu.VMEM((32,), jnp.int32),
        pltpu.VMEM((128,), jnp.float32),
        pltpu.VMEM((HOTCAP,), jnp.int32),
        pltpu.VMEM((128,), jnp.int32),
        pltpu.VMEM((128, CHUNK), jnp.float32),
        pltpu.VMEM((CBUF,), jnp.float32),
        pltpu.VMEM((CBUF,), jnp.int32),
        pltpu.VMEM((CBUF,), jnp.float32),
        pltpu.VMEM((RPW,), jnp.float32),
        pltpu.SemaphoreType.DMA,
        pltpu.SemaphoreType.DMA,
        pltpu.SemaphoreType.DMA,
    ],
)
def _sc_select(d_hbm, m_hbm, t_hbm, vals_hbm, out_hbm,
               mbuf, midx, t_buf, hot_buf, gidx, dbuf, cand_d, cand_i,
               vbuf, obuf, gsem, vsem, msem):
    wid = lax.axis_index("s") * 2 + lax.axis_index("c")
    base_row = wid * RPW
    iota16 = lax.iota(jnp.int32, 16)
    inf16 = jnp.full((16,), INF, jnp.float32)

    pltpu.sync_copy(t_hbm.at[pl.ds(base_row, RPW)], t_buf.at[pl.ds(0, RPW)])
    iw0 = lax.iota(jnp.int32, 16) * (B // 4) + (base_row >> 2)
    iw1 = jnp.minimum(lax.iota(jnp.int32, 16) + 16,
                      jnp.int32(PAD // KB - 1)) * (B // 4) + (base_row >> 2)
    midx[pl.ds(0, 16)] = iw0
    midx[pl.ds(16, 16)] = iw1
    pltpu.async_copy(m_hbm.at[midx], mbuf.at[0], msem).wait()
    for v in range(HOTCAP // 16):
        hot_buf[pl.ds(v * 16, 16)] = jnp.zeros((16,), jnp.int32)
    for v in range(CBUF // 16):
        cand_i[pl.ds(v * 16, 16)] = jnp.zeros((16,), jnp.int32)

    def row_body(b_local, carry):
        o0, o1 = carry
        b = base_row + b_local
        par = b_local & 1
        nxt = jnp.minimum(b_local + 1, RPW - 1)
        i0 = lax.iota(jnp.int32, 16) * (B // 4) + ((base_row + nxt) >> 2)
        i1 = jnp.minimum(lax.iota(jnp.int32, 16) + 16,
                         jnp.int32(PAD // KB - 1)) * (B // 4) + ((base_row + nxt) >> 2)
        midx[pl.ds(0, 16)] = i0
        midx[pl.ds(16, 16)] = i1
        pltpu.async_copy(m_hbm.at[midx], mbuf.at[1 - par], msem)

        t0 = t_buf[pl.ds(0, 16)]
        t1 = t_buf[pl.ds(16, 16)]
        bl16 = jnp.full((16,), b_local, jnp.int32)
        tsel = jnp.where(bl16 < 16, t0, t1)
        tb = _dyn_gather(tsel, bl16 & 15)

        roff = (b & 3) * 32
        ptr_h = jnp.int32(0)
        for v in range(NCHUNK // 16):
            mv = mbuf[par, v >> 1, pl.ds(roff + (v & 1) * 16, 16)]
            msk = mv <= tb
            ph = jnp.minimum(ptr_h, jnp.int32(HOTCAP - 16))
            plsc.store_compressed(hot_buf.at[pl.ds(ph, 16)],
                                  iota16 + (v * 16), mask=msk)
            ptr_h = ptr_h + plsc.all_reduce_population_count(msk)[0]
        nhot = jnp.minimum(ptr_h, jnp.int32(HOTCAP))

        for v in range(CBUF // 16):
            cand_d[pl.ds(v * 16, 16)] = inf16

        nchunks = lax.div(nhot + 127, jnp.int32(128))
        row_off = jnp.full((16,), b * NCHUNK, jnp.int32)

        def chunk_body(c, ptr_c):
            cbase = c * 128
            for v in range(8):
                hv = hot_buf[pl.ds(cbase + v * 16, 16)]
                gidx[pl.ds(v * 16, 16)] = hv + row_off
            pltpu.async_copy(d_hbm.at[gidx], dbuf, gsem).wait()
            jmax = jnp.minimum(nhot - cbase, jnp.int32(128))

            def jbody(j, p):
                hv16 = hot_buf[pl.ds(cbase + (j & ~15), 16)]
                hj = _dyn_gather(hv16, jnp.full((16,), j & 15, jnp.int32))
                kbase = hj * CHUNK
                for v in range(8):
                    dv = dbuf[j, pl.ds(v * 16, 16)]
                    msk = dv <= tb
                    gk = kbase + (iota16 + v * 16)
                    ps = jnp.minimum(p, jnp.int32(CCAP))
                    plsc.store_compressed(cand_d.at[pl.ds(ps, 16)],
                                          dv, mask=msk)
                    plsc.store_compressed(cand_i.at[pl.ds(ps, 16)],
                                          gk, mask=msk)
                    p = p + plsc.all_reduce_population_count(msk)[0]
                return p

            return lax.fori_loop(0, jmax, jbody, ptr_c)

        nc = jnp.minimum(lax.fori_loop(0, nchunks, chunk_body, jnp.int32(0)),
                         jnp.int32(CCAP))
        nv = lax.div(nc + 15, jnp.int32(16))

        def vg(vv, _):
            pltpu.async_copy(vals_hbm.at[cand_i.at[pl.ds(vv * 16, 16)]],
                             vbuf.at[pl.ds(vv * 16, 16)], vsem)
            return 0
        lax.fori_loop(0, nv, vg, 0)

        def excl(_, prev):
            pv = jnp.full((16,), prev, jnp.int32)

            def scan1(vv, rm):
                dv = cand_d[pl.ds(vv * 16, 16)]
                iv = cand_i[pl.ds(vv * 16, 16)]
                dv = jnp.where((iv == pv) & (dv < BIGF), -1.0, dv)
                cand_d[pl.ds(vv * 16, 16)] = dv
                return jnp.maximum(rm, jnp.where(dv < BIGF, dv, -1.0))

            rm = lax.fori_loop(0, nv, scan1, jnp.full((16,), -1.0))
            m = jnp.max(rm)
            mv16 = jnp.full((16,), m, jnp.float32)

            def scan2(vv, r2):
                dv = cand_d[pl.ds(vv * 16, 16)]
                iv = cand_i[pl.ds(vv * 16, 16)]
                return jnp.maximum(
                    r2, jnp.where((dv == mv16) & (dv < BIGF), iv, -1))

            r2 = lax.fori_loop(0, nv, scan2, jnp.full((16,), -1, jnp.int32))
            return jnp.max(r2)

        last = lax.fori_loop(0, nc - K, excl, jnp.int32(-1))
        lastv = jnp.full((16,), last, jnp.int32)

        def vd(vv, _):
            pltpu.make_async_copy(
                vals_hbm.at[cand_i.at[pl.ds(vv * 16, 16)]],
                vbuf.at[pl.ds(vv * 16, 16)], vsem).wait()
            return 0
        lax.fori_loop(0, nv, vd, 0)

        def acc_body(vv, acc):
            sv = cand_d[pl.ds(vv * 16, 16)]
            iv = cand_i[pl.ds(vv * 16, 16)]
            vv16 = vbuf[pl.ds(vv * 16, 16)]
            sv = jnp.where(iv == lastv, -1.0, sv)
            keep = (sv >= 0.0) & (sv < BIGF)
            sc = jnp.where(keep, sv, 1.0)
            y = plsc.bitcast(
                0x5F3759DF - (plsc.bitcast(sc, jnp.int32) >> 1), jnp.float32)
            for _ in range(3):
                y = y * (1.5 - 0.5 * sc * y * y)
            dv = sc * y
            w = dv / (dv + EPS)
            return acc + jnp.where(keep, w * vv16, 0.0)

        acc = lax.fori_loop(0, nv, acc_body, jnp.zeros((16,), jnp.float32))
        s = jnp.sum(acc) * jnp.float32(1.0 / K)

        pltpu.make_async_copy(m_hbm.at[midx],
                              mbuf.at[1 - par], msem).wait()

        s16 = jnp.full((16,), s, jnp.float32)
        o0 = jnp.where(iota16 == bl16, s16, o0)
        o1 = jnp.where(iota16 == (bl16 - 16), s16, o1)
        return o0, o1

    z = jnp.zeros((16,), jnp.float32)
    o0, o1 = lax.fori_loop(0, RPW, row_body, (z, z))
    obuf[pl.ds(0, 16)] = o0
    obuf[pl.ds(16, 16)] = o1
    pltpu.sync_copy(obuf, out_hbm.at[pl.ds(base_row, RPW)])


@jax.jit
def kernel(queries, keys, values):
    keys_p = jnp.concatenate(
        [keys, jnp.full((PAD - CAP, D), 1e4, jnp.float32)], axis=0)
    d, m3 = _tc_stage(queries, keys_p)
    t2 = _t2_stage(m3)
    vals_p = jnp.concatenate(
        [values[:, 0], jnp.zeros((PAD - CAP,), jnp.float32)], axis=0)
    out = _sc_select(
        jnp.reshape(d, (B * NCHUNK, CHUNK)),
        jnp.reshape(m3, ((PAD // KB) * (B // 4), 128)),
        jnp.reshape(t2, (B,)),
        vals_p,
    )
    return jnp.reshape(out, (B, 1))


# speedup vs baseline: 1.0292x; 1.0292x over previous
import functools

import jax
import jax.numpy as jnp
from jax import lax
from jax.experimental import pallas as pl
from jax.experimental.pallas import tpu as pltpu
from jax.experimental.pallas import tpu_sc as plsc

CAP = 100000
PAD = 102400
D = 64
K = 32
B = 1024
EPS = 1e-08

QT = 128
KB = 4096
CHUNK = 128
NCHUNK = PAD // CHUNK
CPB = KB // CHUNK
MID = 1024
NMID = PAD // MID
MPB = KB // MID

NW = 32
RPW = B // NW
HOTCAP = 640
CCAP = 512
CBUF = CCAP + 32
INF = float("inf")
BIGF = 1e30


def _dist_kernel(q_ref, k_ref, qn_ref, kn_ref, d_ref, m_ref):
    q = q_ref[...]
    k = k_ref[...]
    sq = qn_ref[...] + kn_ref[...] - 2.0 * lax.dot_general(
        q, k, (((1,), (1,)), ((), ())),
        preferred_element_type=jnp.float32,
        precision=lax.Precision.DEFAULT)
    s = jnp.maximum(sq, 1e-12)
    d_ref[...] = s
    m_ref[0, ...] = jnp.min(s.reshape(QT, CPB, CHUNK), axis=2)


def _tc_stage(queries, keys_p):
    qn = jnp.sum(queries * queries, axis=1, keepdims=True)
    kn = jnp.sum(keys_p * keys_p, axis=1, keepdims=True).T
    grid = (B // QT, PAD // KB)
    return pl.pallas_call(
        _dist_kernel,
        grid=grid,
        in_specs=[
            pl.BlockSpec((QT, D), lambda i, j: (i, 0)),
            pl.BlockSpec((KB, D), lambda i, j: (j, 0)),
            pl.BlockSpec((QT, 1), lambda i, j: (i, 0)),
            pl.BlockSpec((1, KB), lambda i, j: (0, j)),
        ],
        out_specs=[
            pl.BlockSpec((QT, KB), lambda i, j: (i, j)),
            pl.BlockSpec((1, QT, CPB), lambda i, j: (j, i, 0)),
        ],
        out_shape=[
            jax.ShapeDtypeStruct((B, PAD), jnp.float32),
            jax.ShapeDtypeStruct((PAD // KB, B, CPB), jnp.float32),
        ],
    )(queries, keys_p, qn, kn)


def _t2_kernel(m_ref, t2_ref):
    x = m_ref[...]
    m = jnp.min(x, axis=1, keepdims=True)
    for _ in range(K - 1):
        x = jnp.where(x == m, INF, x)
        m = jnp.min(x, axis=1, keepdims=True)
    t2_ref[...] = m


def _t2_stage(m):
    return pl.pallas_call(
        _t2_kernel,
        grid=(B // QT,),
        in_specs=[pl.BlockSpec((QT, NCHUNK), lambda i: (i, 0))],
        out_specs=pl.BlockSpec((QT, 1), lambda i: (i, 0)),
        out_shape=jax.ShapeDtypeStruct((B, 1), jnp.float32),
    )(m)


def _dyn_gather(x, idx16):
    return lax.gather(
        x, idx16[:, None],
        lax.GatherDimensionNumbers(
            offset_dims=(), collapsed_slice_dims=(0,), start_index_map=(0,)),
        (1,), mode=lax.GatherScatterMode.PROMISE_IN_BOUNDS)


_MESH = plsc.VectorSubcoreMesh(core_axis_name="c", subcore_axis_name="s")


@functools.partial(
    pl.kernel,
    mesh=_MESH,
    compiler_params=pltpu.CompilerParams(needs_layout_passes=False),
    out_type=jax.ShapeDtypeStruct((B,), jnp.float32),
    scratch_types=[
        pltpu.VMEM((2, NCHUNK), jnp.float32),
        pltpu.VMEM((128,), jnp.float32),
        pltpu.VMEM((HOTCAP,), jnp.int32),
        pltpu.VMEM((128,), jnp.int32),
        pltpu.VMEM((128, CHUNK), jnp.float32),
        pltpu.VMEM((CBUF,), jnp.float32),
        pltpu.VMEM((CBUF,), jnp.int32),
        pltpu.VMEM((CBUF,), jnp.float32),
        pltpu.VMEM((RPW,), jnp.float32),
        pltpu.SemaphoreType.DMA,
        pltpu.SemaphoreType.DMA,
        pltpu.SemaphoreType.DMA,
    ],
)
def _sc_select(d_hbm, m_hbm, t_hbm, vals_hbm, out_hbm,
               mbuf, t_buf, hot_buf, gidx, dbuf, cand_d, cand_i,
               vbuf, obuf, gsem, vsem, msem):
    wid = lax.axis_index("s") * 2 + lax.axis_index("c")
    base_row = wid * RPW
    iota16 = lax.iota(jnp.int32, 16)
    inf16 = jnp.full((16,), INF, jnp.float32)

    pltpu.sync_copy(t_hbm.at[pl.ds(base_row, RPW)], t_buf.at[pl.ds(0, RPW)])
    pltpu.sync_copy(m_hbm.at[base_row], mbuf.at[0])
    for v in range(HOTCAP // 16):
        hot_buf[pl.ds(v * 16, 16)] = jnp.zeros((16,), jnp.int32)
    for v in range(CBUF // 16):
        cand_i[pl.ds(v * 16, 16)] = jnp.zeros((16,), jnp.int32)

    def row_body(b_local, carry):
        o0, o1 = carry
        b = base_row + b_local
        par = b_local & 1
        nxt = jnp.minimum(b_local + 1, RPW - 1)
        pltpu.async_copy(m_hbm.at[base_row + nxt], mbuf.at[1 - par], msem)

        t0 = t_buf[pl.ds(0, 16)]
        t1 = t_buf[pl.ds(16, 16)]
        bl16 = jnp.full((16,), b_local, jnp.int32)
        tsel = jnp.where(bl16 < 16, t0, t1)
        tb = _dyn_gather(tsel, bl16 & 15)

        ptr_h = jnp.int32(0)
        for v in range(NCHUNK // 16):
            mv = mbuf[par, pl.ds(v * 16, 16)]
            msk = mv <= tb
            ph = jnp.minimum(ptr_h, jnp.int32(HOTCAP - 16))
            plsc.store_compressed(hot_buf.at[pl.ds(ph, 16)],
                                  iota16 + (v * 16), mask=msk)
            ptr_h = ptr_h + plsc.all_reduce_population_count(msk)[0]
        nhot = jnp.minimum(ptr_h, jnp.int32(HOTCAP))

        for v in range(CBUF // 16):
            cand_d[pl.ds(v * 16, 16)] = inf16

        nchunks = lax.div(nhot + 127, jnp.int32(128))
        row_off = jnp.full((16,), b * NCHUNK, jnp.int32)

        def chunk_body(c, ptr_c):
            cbase = c * 128
            for v in range(8):
                hv = hot_buf[pl.ds(cbase + v * 16, 16)]
                gidx[pl.ds(v * 16, 16)] = hv + row_off
            pltpu.async_copy(d_hbm.at[gidx], dbuf, gsem).wait()
            jmax = jnp.minimum(nhot - cbase, jnp.int32(128))

            def jbody(j, p):
                hv16 = hot_buf[pl.ds(cbase + (j & ~15), 16)]
                hj = _dyn_gather(hv16, jnp.full((16,), j & 15, jnp.int32))
                kbase = hj * CHUNK
                for v in range(8):
                    dv = dbuf[j, pl.ds(v * 16, 16)]
                    msk = dv <= tb
                    gk = kbase + (iota16 + v * 16)
                    ps = jnp.minimum(p, jnp.int32(CCAP))
                    plsc.store_compressed(cand_d.at[pl.ds(ps, 16)],
                                          dv, mask=msk)
                    plsc.store_compressed(cand_i.at[pl.ds(ps, 16)],
                                          gk, mask=msk)
                    p = p + plsc.all_reduce_population_count(msk)[0]
                return p

            return lax.fori_loop(0, jmax, jbody, ptr_c)

        nc = jnp.minimum(lax.fori_loop(0, nchunks, chunk_body, jnp.int32(0)),
                         jnp.int32(CCAP))
        nv = lax.div(nc + 15, jnp.int32(16))

        def vg(vv, _):
            pltpu.async_copy(vals_hbm.at[cand_i.at[pl.ds(vv * 16, 16)]],
                             vbuf.at[pl.ds(vv * 16, 16)], vsem)
            return 0
        lax.fori_loop(0, nv, vg, 0)

        def excl(_, prev):
            pv = jnp.full((16,), prev, jnp.int32)

            def scan1(vv, rm):
                dv = cand_d[pl.ds(vv * 16, 16)]
                iv = cand_i[pl.ds(vv * 16, 16)]
                dv = jnp.where((iv == pv) & (dv < BIGF), -1.0, dv)
                cand_d[pl.ds(vv * 16, 16)] = dv
                return jnp.maximum(rm, jnp.where(dv < BIGF, dv, -1.0))

            rm = lax.fori_loop(0, nv, scan1, jnp.full((16,), -1.0))
            m = jnp.max(rm)
            mv16 = jnp.full((16,), m, jnp.float32)

            def scan2(vv, r2):
                dv = cand_d[pl.ds(vv * 16, 16)]
                iv = cand_i[pl.ds(vv * 16, 16)]
                return jnp.maximum(
                    r2, jnp.where((dv == mv16) & (dv < BIGF), iv, -1))

            r2 = lax.fori_loop(0, nv, scan2, jnp.full((16,), -1, jnp.int32))
            return jnp.max(r2)

        last = lax.fori_loop(0, nc - K, excl, jnp.int32(-1))
        lastv = jnp.full((16,), last, jnp.int32)

        def vd(vv, _):
            pltpu.make_async_copy(
                vals_hbm.at[cand_i.at[pl.ds(vv * 16, 16)]],
                vbuf.at[pl.ds(vv * 16, 16)], vsem).wait()
            return 0
        lax.fori_loop(0, nv, vd, 0)

        def acc_body(vv, acc):
            sv = cand_d[pl.ds(vv * 16, 16)]
            iv = cand_i[pl.ds(vv * 16, 16)]
            vv16 = vbuf[pl.ds(vv * 16, 16)]
            sv = jnp.where(iv == lastv, -1.0, sv)
            keep = (sv >= 0.0) & (sv < BIGF)
            sc = jnp.where(keep, sv, 1.0)
            y = plsc.bitcast(
                0x5F3759DF - (plsc.bitcast(sc, jnp.int32) >> 1), jnp.float32)
            for _ in range(3):
                y = y * (1.5 - 0.5 * sc * y * y)
            dv = sc * y
            w = dv / (dv + EPS)
            return acc + jnp.where(keep, w * vv16, 0.0)

        acc = lax.fori_loop(0, nv, acc_body, jnp.zeros((16,), jnp.float32))
        s = jnp.sum(acc) * jnp.float32(1.0 / K)

        pltpu.make_async_copy(m_hbm.at[base_row + nxt],
                              mbuf.at[1 - par], msem).wait()

        s16 = jnp.full((16,), s, jnp.float32)
        o0 = jnp.where(iota16 == bl16, s16, o0)
        o1 = jnp.where(iota16 == (bl16 - 16), s16, o1)
        return o0, o1

    z = jnp.zeros((16,), jnp.float32)
    o0, o1 = lax.fori_loop(0, RPW, row_body, (z, z))
    obuf[pl.ds(0, 16)] = o0
    obuf[pl.ds(16, 16)] = o1
    pltpu.sync_copy(obuf, out_hbm.at[pl.ds(base_row, RPW)])


@jax.jit
def kernel(queries, keys, values):
    keys_p = jnp.concatenate(
        [keys, jnp.full((PAD - CAP, D), 1e4, jnp.float32)], axis=0)
    d, m3 = _tc_stage(queries, keys_p)
    m = jnp.transpose(m3, (1, 0, 2)).reshape(B, NCHUNK)
    t2 = _t2_stage(m)
    vals_p = jnp.concatenate(
        [values[:, 0], jnp.zeros((PAD - CAP,), jnp.float32)], axis=0)
    out = _sc_select(
        jnp.reshape(d, (B * NCHUNK, CHUNK)),
        m,
        jnp.reshape(t2, (B,)),
        vals_p,
    )
    return jnp.reshape(out, (B, 1))
